# baseline (speedup 1.0000x reference)
import functools

import jax
import jax.numpy as jnp
from jax import lax
from jax.experimental import pallas as pl
from jax.experimental.pallas import tpu as pltpu
from jax.experimental.pallas import tpu_sc as plsc

N_NODES = 10000
N_EDGES = 160000
N_GRAPHS = 128
HIDDEN = 256
N_CLASSES = 10

NSUB = 16
NPAD = 10240
NT = NPAD // NSUB
CHUNK = 128
EC = 80
KW = 16
NB = EC // KW
EPT = EC * CHUNK
EPAD = NSUB * EPT
PAD_NODE = NPAD - 1
GCHUNK = 128
GC = NT // GCHUNK
PAD_GID = N_GRAPHS
GBUF = 136


def _rsqrt16(m):
    i = lax.bitcast_convert_type(m, jnp.int32)
    i = jnp.int32(0x5F3759DF) - (i >> 1)
    x = lax.bitcast_convert_type(i, jnp.float32)
    for _ in range(3):
        x = x * (1.5 - 0.5 * m * x * x)
    return x


def _sc_body(src_hbm, dst_hbm, gid_hbm, out_hbm,
             ya, aa, g0, g1, g2, g3,
             src_v, dst_v, gid_v, vals, ones_v, zeros_v,
             la, norm_v, n2_v, keep1, keep2, sem_g, sem_s):
    cid = lax.axis_index("c")
    tid = lax.axis_index("s")
    is0 = cid == 0

    one16 = jnp.full((16,), 1.0, jnp.float32)
    zero16 = jnp.zeros((16,), jnp.float32)
    for i in range(CHUNK // 16):
        ones_v[pl.ds(i * 16, 16)] = one16
    for i in range(NT // 16):
        zeros_v[pl.ds(i * 16, 16)] = zero16

    pltpu.sync_copy(src_hbm.at[tid], src_v)
    pltpu.sync_copy(dst_hbm.at[tid], dst_v)
    pltpu.sync_copy(gid_hbm.at[tid], gid_v)

    nsl = pl.ds(tid * NT, NT)

    pltpu.sync_copy(zeros_v, aa.at[nsl])

    @pl.when(tid == 0)
    def _():
        z = zeros_v.at[pl.ds(0, GBUF)]
        pltpu.sync_copy(z, g0)
        pltpu.sync_copy(z, g1)
        pltpu.sync_copy(z, g2)
        pltpu.sync_copy(z, g3)

    plsc.subcore_barrier()

    def fire_gathers(b):
        def f(k, c):
            j = b * KW + k
            pltpu.async_copy(ya.at[src_v.at[j]], vals.at[j], sem_g)
            return c
        lax.fori_loop(0, KW, f, 0)

    def drain_gathers(b):
        def f(k, c):
            j = b * KW + k
            pltpu.make_async_copy(ya.at[src_v.at[j]], vals.at[j], sem_g).wait()
            return c
        lax.fori_loop(0, KW, f, 0)

    def fire_scatters(b, valrow):
        def f(k, c):
            j = b * KW + k
            pltpu.async_copy(valrow(j), aa.at[dst_v.at[j]], sem_s, add=True)
            return c
        lax.fori_loop(0, KW, f, 0)

    def drain_scatters(b, valrow):
        def f(k, c):
            j = b * KW + k
            pltpu.make_async_copy(valrow(j), aa.at[dst_v.at[j]], sem_s).wait()
            return c
        lax.fori_loop(0, KW, f, 0)

    ones_row = lambda j: ones_v

    def deg_blk(b, c):
        @pl.when(b > 1)
        def _():
            drain_scatters(b - 2, ones_row)
        fire_scatters(b, ones_row)
        return c

    lax.fori_loop(0, NB, deg_blk, 0)
    drain_scatters(NB - 2, ones_row)
    drain_scatters(NB - 1, ones_row)
    plsc.subcore_barrier()

    def edge_pass():
        vrow = lambda j: vals.at[j]
        fire_gathers(0)
        fire_gathers(1)

        def blk(b, c):
            drain_gathers(b)

            @pl.when(b + 2 < NB)
            def _():
                fire_gathers(b + 2)

            @pl.when(b > 1)
            def _():
                drain_scatters(b - 2, vrow)

            fire_scatters(b, vrow)
            return c

        lax.fori_loop(0, NB, blk, 0)
        drain_scatters(NB - 2, vrow)
        drain_scatters(NB - 1, vrow)

    pltpu.sync_copy(aa.at[nsl], la)
    for i in range(NT // 16):
        s16 = pl.ds(i * 16, 16)
        d = la[s16]
        n = _rsqrt16(jnp.maximum(d, 1.0))
        norm_v[s16] = n
        n2_v[s16] = n * n
        la[s16] = jnp.where(is0, d * n, n)
    pltpu.sync_copy(la, ya.at[nsl])
    pltpu.sync_copy(zeros_v, aa.at[nsl])
    plsc.subcore_barrier()

    edge_pass()
    plsc.subcore_barrier()

    pltpu.sync_copy(aa.at[nsl], la)
    for i in range(NT // 16):
        s16 = pl.ds(i * 16, 16)
        v = la[s16]
        keep1[s16] = norm_v[s16] * v
        la[s16] = n2_v[s16] * v
    pltpu.sync_copy(la, ya.at[nsl])
    pltpu.sync_copy(zeros_v, aa.at[nsl])
    plsc.subcore_barrier()

    edge_pass()
    plsc.subcore_barrier()

    pltpu.sync_copy(aa.at[nsl], la)
    for i in range(NT // 16):
        s16 = pl.ds(i * 16, 16)
        v = la[s16]
        keep2[s16] = norm_v[s16] * v
        la[s16] = n2_v[s16] * v

    @pl.when(is0)
    def _():
        pltpu.sync_copy(la, ya.at[nsl])
        pltpu.sync_copy(zeros_v, aa.at[nsl])
        plsc.subcore_barrier()
        edge_pass()
        plsc.subcore_barrier()
        pltpu.sync_copy(aa.at[nsl], la)
        for i in range(NT // 16):
            s16 = pl.ds(i * 16, 16)
            keep2[s16] = norm_v[s16] * la[s16]

    def pool_scatter(valref, gbuf):
        def body(j, c):
            off = pl.multiple_of(j * GCHUNK, GCHUNK)
            pltpu.sync_copy(valref.at[pl.ds(off, GCHUNK)],
                            gbuf.at[gid_v.at[j]], add=True)
            return c

        lax.fori_loop(0, GC, body, 0)

    @pl.when(is0)
    def _():
        pool_scatter(keep2, g0)

    @pl.when(jnp.logical_not(is0))
    def _():
        pool_scatter(keep2, g1)
        pool_scatter(keep1, g2)

        def cnt_body(j, c):
            pltpu.sync_copy(ones_v.at[pl.ds(0, GCHUNK)],
                            g3.at[gid_v.at[j]], add=True)
            return c

        lax.fori_loop(0, GC, cnt_body, 0)

    plsc.subcore_barrier()

    @pl.when(jnp.logical_and(is0, tid == 0))
    def _():
        pltpu.sync_copy(g0.at[pl.ds(0, N_GRAPHS)], out_hbm.at[0])

    @pl.when(jnp.logical_and(jnp.logical_not(is0), tid == 0))
    def _():
        pltpu.sync_copy(g1.at[pl.ds(0, N_GRAPHS)], out_hbm.at[1])
        pltpu.sync_copy(g2.at[pl.ds(0, N_GRAPHS)], out_hbm.at[2])
        pltpu.sync_copy(g3.at[pl.ds(0, N_GRAPHS)], out_hbm.at[3])


_sc_sums = pl.kernel(
    _sc_body,
    out_type=jax.ShapeDtypeStruct((4, N_GRAPHS), jnp.float32),
    mesh=plsc.VectorSubcoreMesh(core_axis_name="c", subcore_axis_name="s"),
    scratch_types=[
        pltpu.VMEM_SHARED((NPAD,), jnp.float32),
        pltpu.VMEM_SHARED((NPAD,), jnp.float32),
        pltpu.VMEM_SHARED((GBUF,), jnp.float32),
        pltpu.VMEM_SHARED((GBUF,), jnp.float32),
        pltpu.VMEM_SHARED((GBUF,), jnp.float32),
        pltpu.VMEM_SHARED((GBUF,), jnp.float32),
        pltpu.VMEM((EC, CHUNK), jnp.int32),
        pltpu.VMEM((EC, CHUNK), jnp.int32),
        pltpu.VMEM((GC, GCHUNK), jnp.int32),
        pltpu.VMEM((EC, CHUNK), jnp.float32),

        pltpu.VMEM((CHUNK,), jnp.float32),
        pltpu.VMEM((NT,), jnp.float32),
        pltpu.VMEM((NT,), jnp.float32),
        pltpu.VMEM((NT,), jnp.float32),
        pltpu.VMEM((NT,), jnp.float32),
        pltpu.VMEM((NT,), jnp.float32),
        pltpu.VMEM((NT,), jnp.float32),
        pltpu.SemaphoreType.DMA,
        pltpu.SemaphoreType.DMA,
    ],
)


def _combine_body(sums_ref, W0_ref, b0_ref, W1_ref, b1_ref, W2_ref, b2_ref,
                  Wc_ref, bc_ref, out_ref):
    f32 = jnp.float32
    sums = sums_ref[...]
    cnt = sums[3:4, :]
    inv = 1.0 / jnp.maximum(cnt, 1.0)
    M4 = sums * inv

    W1 = W1_ref[...]
    W2 = W2_ref[...]
    Wc = Wc_ref[...]
    A = jnp.dot(jnp.dot(W0_ref[...], W1, preferred_element_type=f32, precision=lax.Precision.HIGHEST), W2,
                preferred_element_type=f32, precision=lax.Precision.HIGHEST)
    B = jnp.dot(jnp.dot(b0_ref[...], W1, preferred_element_type=f32, precision=lax.Precision.HIGHEST), W2,
                preferred_element_type=f32, precision=lax.Precision.HIGHEST)
    C = jnp.dot(b1_ref[...], W2, preferred_element_type=f32, precision=lax.Precision.HIGHEST)
    V = jnp.concatenate([
        jnp.dot(A, Wc, preferred_element_type=f32, precision=lax.Precision.HIGHEST),
        jnp.dot(B, Wc, preferred_element_type=f32, precision=lax.Precision.HIGHEST),
        jnp.dot(C, Wc, preferred_element_type=f32, precision=lax.Precision.HIGHEST),
        jnp.dot(b2_ref[...], Wc, preferred_element_type=f32, precision=lax.Precision.HIGHEST),
    ], axis=0)
    out = lax.dot_general(M4, V, (((0,), (0,)), ((), ())),
                          preferred_element_type=f32, precision=lax.Precision.HIGHEST)
    out_ref[...] = out + bc_ref[...]


_combine = pl.pallas_call(
    _combine_body,
    out_shape=jax.ShapeDtypeStruct((N_GRAPHS, N_CLASSES), jnp.float32),
)


@jax.jit
def kernel(edge_index, node_graph_ids, W0, b0, W1, b1, W2, b2, Wc, bc):
    src = edge_index[0]
    dst = edge_index[1]
    pad = jnp.full((EPAD - N_EDGES,), PAD_NODE, jnp.int32)
    src3 = jnp.concatenate([src, pad]).reshape(NSUB, EC, CHUNK)
    dst3 = jnp.concatenate([dst, pad]).reshape(NSUB, EC, CHUNK)
    gpad = jnp.full((NPAD - N_NODES,), PAD_GID, jnp.int32)
    gid3 = jnp.concatenate([node_graph_ids.astype(jnp.int32),
                            gpad]).reshape(NSUB, GC, GCHUNK)

    sums = _sc_sums(src3, dst3, gid3)
    return _combine(sums, W0, b0.reshape(1, HIDDEN), W1, b1.reshape(1, HIDDEN),
                    W2, b2.reshape(1, HIDDEN), Wc, bc.reshape(1, N_CLASSES))


# speedup vs baseline: 1.1262x; 1.1262x over previous
import functools

import jax
import jax.numpy as jnp
from jax import lax
from jax.experimental import pallas as pl
from jax.experimental.pallas import tpu as pltpu
from jax.experimental.pallas import tpu_sc as plsc

N_NODES = 10000
N_EDGES = 160000
N_GRAPHS = 128
HIDDEN = 256
N_CLASSES = 10

NSUB = 16
NPAD = 10240
NT = NPAD // NSUB
CHUNK = 128
EC = 80
KW = 16
NB = EC // KW
EPT = EC * CHUNK
EPAD = NSUB * EPT
PAD_NODE = NPAD - 1
GCHUNK = 128
GC = NT // GCHUNK
PAD_GID = N_GRAPHS
GBUF = 136


def _rsqrt16(m):
    i = lax.bitcast_convert_type(m, jnp.int32)
    i = jnp.int32(0x5F3759DF) - (i >> 1)
    x = lax.bitcast_convert_type(i, jnp.float32)
    for _ in range(3):
        x = x * (1.5 - 0.5 * m * x * x)
    return x


def _sc_body(edges_hbm, gid_hbm, out_hbm,
             ya, aa, g0, g1, g2, g3,
             src_v, dst_v, gid_v, vals, ones_v, zeros_v,
             la, norm_v, n2_v, keep1, keep2, sem_g, sem_s):
    cid = lax.axis_index("c")
    tid = lax.axis_index("s")
    is0 = cid == 0

    one16 = jnp.full((16,), 1.0, jnp.float32)
    zero16 = jnp.zeros((16,), jnp.float32)
    for i in range(CHUNK // 16):
        ones_v[pl.ds(i * 16, 16)] = one16
    for i in range(NT // 16):
        zeros_v[pl.ds(i * 16, 16)] = zero16

    pltpu.sync_copy(edges_hbm.at[0, tid], src_v)
    pltpu.sync_copy(edges_hbm.at[1, tid], dst_v)
    pltpu.sync_copy(gid_hbm.at[tid], gid_v)

    nsl = pl.ds(tid * NT, NT)

    pltpu.sync_copy(zeros_v, aa.at[nsl])

    @pl.when(tid == 0)
    def _():
        z = zeros_v.at[pl.ds(0, GBUF)]
        pltpu.sync_copy(z, g0)
        pltpu.sync_copy(z, g1)
        pltpu.sync_copy(z, g2)
        pltpu.sync_copy(z, g3)

    plsc.subcore_barrier()

    def fire_gathers(b):
        def f(k, c):
            j = b * KW + k
            pltpu.async_copy(ya.at[src_v.at[j]], vals.at[j], sem_g)
            return c
        lax.fori_loop(0, KW, f, 0)

    def drain_gathers(b):
        def f(k, c):
            j = b * KW + k
            pltpu.make_async_copy(ya.at[src_v.at[j]], vals.at[j], sem_g).wait()
            return c
        lax.fori_loop(0, KW, f, 0)

    def fire_scatters(b, valrow):
        def f(k, c):
            j = b * KW + k
            pltpu.async_copy(valrow(j), aa.at[dst_v.at[j]], sem_s, add=True)
            return c
        lax.fori_loop(0, KW, f, 0)

    def drain_scatters(b, valrow):
        def f(k, c):
            j = b * KW + k
            pltpu.make_async_copy(valrow(j), aa.at[dst_v.at[j]], sem_s).wait()
            return c
        lax.fori_loop(0, KW, f, 0)

    ones_row = lambda j: ones_v

    def deg_blk(b, c):
        @pl.when(b > 0)
        def _():
            drain_scatters(b - 1, ones_row)
        fire_scatters(b, ones_row)
        return c

    lax.fori_loop(0, NB, deg_blk, 0)
    drain_scatters(NB - 1, ones_row)
    plsc.subcore_barrier()

    def edge_pass():
        vrow = lambda j: vals.at[j]
        fire_gathers(0)

        def blk(b, c):
            drain_gathers(b)

            @pl.when(b + 1 < NB)
            def _():
                fire_gathers(b + 1)

            @pl.when(b > 0)
            def _():
                drain_scatters(b - 1, vrow)

            fire_scatters(b, vrow)
            return c

        lax.fori_loop(0, NB, blk, 0)
        drain_scatters(NB - 1, vrow)

    pltpu.sync_copy(aa.at[nsl], la)
    for i in range(NT // 16):
        s16 = pl.ds(i * 16, 16)
        d = la[s16]
        n = _rsqrt16(jnp.maximum(d, 1.0))
        norm_v[s16] = n
        n2_v[s16] = n * n
        la[s16] = jnp.where(is0, d * n, n)
    pltpu.sync_copy(la, ya.at[nsl])
    pltpu.sync_copy(zeros_v, aa.at[nsl])
    plsc.subcore_barrier()

    edge_pass()
    plsc.subcore_barrier()

    pltpu.sync_copy(aa.at[nsl], la)
    for i in range(NT // 16):
        s16 = pl.ds(i * 16, 16)
        v = la[s16]
        keep1[s16] = norm_v[s16] * v
        la[s16] = n2_v[s16] * v
    pltpu.sync_copy(la, ya.at[nsl])
    pltpu.sync_copy(zeros_v, aa.at[nsl])
    plsc.subcore_barrier()

    edge_pass()
    plsc.subcore_barrier()

    pltpu.sync_copy(aa.at[nsl], la)
    for i in range(NT // 16):
        s16 = pl.ds(i * 16, 16)
        v = la[s16]
        keep2[s16] = norm_v[s16] * v
        la[s16] = n2_v[s16] * v

    @pl.when(is0)
    def _():
        pltpu.sync_copy(la, ya.at[nsl])
        pltpu.sync_copy(zeros_v, aa.at[nsl])
        plsc.subcore_barrier()
        edge_pass()
        plsc.subcore_barrier()
        pltpu.sync_copy(aa.at[nsl], la)
        for i in range(NT // 16):
            s16 = pl.ds(i * 16, 16)
            keep2[s16] = norm_v[s16] * la[s16]

    def pool_scatter(valref, gbuf):
        def body(j, c):
            off = pl.multiple_of(j * GCHUNK, GCHUNK)
            pltpu.sync_copy(valref.at[pl.ds(off, GCHUNK)],
                            gbuf.at[gid_v.at[j]], add=True)
            return c

        lax.fori_loop(0, GC, body, 0)

    @pl.when(is0)
    def _():
        pool_scatter(keep2, g0)

    @pl.when(jnp.logical_not(is0))
    def _():
        pool_scatter(keep2, g1)
        pool_scatter(keep1, g2)

        def cnt_body(j, c):
            pltpu.sync_copy(ones_v.at[pl.ds(0, GCHUNK)],
                            g3.at[gid_v.at[j]], add=True)
            return c

        lax.fori_loop(0, GC, cnt_body, 0)

    plsc.subcore_barrier()

    @pl.when(jnp.logical_and(is0, tid == 0))
    def _():
        pltpu.sync_copy(g0.at[pl.ds(0, N_GRAPHS)], out_hbm.at[0])

    @pl.when(jnp.logical_and(jnp.logical_not(is0), tid == 0))
    def _():
        pltpu.sync_copy(g1.at[pl.ds(0, N_GRAPHS)], out_hbm.at[1])
        pltpu.sync_copy(g2.at[pl.ds(0, N_GRAPHS)], out_hbm.at[2])
        pltpu.sync_copy(g3.at[pl.ds(0, N_GRAPHS)], out_hbm.at[3])


_sc_sums = pl.kernel(
    _sc_body,
    out_type=jax.ShapeDtypeStruct((4, N_GRAPHS), jnp.float32),
    mesh=plsc.VectorSubcoreMesh(core_axis_name="c", subcore_axis_name="s"),
    scratch_types=[
        pltpu.VMEM_SHARED((NPAD,), jnp.float32),
        pltpu.VMEM_SHARED((NPAD,), jnp.float32),
        pltpu.VMEM_SHARED((GBUF,), jnp.float32),
        pltpu.VMEM_SHARED((GBUF,), jnp.float32),
        pltpu.VMEM_SHARED((GBUF,), jnp.float32),
        pltpu.VMEM_SHARED((GBUF,), jnp.float32),
        pltpu.VMEM((EC, CHUNK), jnp.int32),
        pltpu.VMEM((EC, CHUNK), jnp.int32),
        pltpu.VMEM((GC, GCHUNK), jnp.int32),
        pltpu.VMEM((EC, CHUNK), jnp.float32),

        pltpu.VMEM((CHUNK,), jnp.float32),
        pltpu.VMEM((NT,), jnp.float32),
        pltpu.VMEM((NT,), jnp.float32),
        pltpu.VMEM((NT,), jnp.float32),
        pltpu.VMEM((NT,), jnp.float32),
        pltpu.VMEM((NT,), jnp.float32),
        pltpu.VMEM((NT,), jnp.float32),
        pltpu.SemaphoreType.DMA,
        pltpu.SemaphoreType.DMA,
    ],
)


def _weights_body(W0_ref, b0_ref, W1_ref, b1_ref, W2_ref, b2_ref,
                  Wc_ref, out_ref):
    f32 = jnp.float32
    W1 = W1_ref[...]
    W2 = W2_ref[...]
    Wc = Wc_ref[...]
    A = jnp.dot(jnp.dot(W0_ref[...], W1, preferred_element_type=f32, precision=lax.Precision.HIGHEST), W2,
                preferred_element_type=f32, precision=lax.Precision.HIGHEST)
    B = jnp.dot(jnp.dot(b0_ref[...], W1, preferred_element_type=f32, precision=lax.Precision.HIGHEST), W2,
                preferred_element_type=f32, precision=lax.Precision.HIGHEST)
    C = jnp.dot(b1_ref[...], W2, preferred_element_type=f32, precision=lax.Precision.HIGHEST)
    V = jnp.concatenate([
        jnp.dot(A, Wc, preferred_element_type=f32, precision=lax.Precision.HIGHEST),
        jnp.dot(B, Wc, preferred_element_type=f32, precision=lax.Precision.HIGHEST),
        jnp.dot(C, Wc, preferred_element_type=f32, precision=lax.Precision.HIGHEST),
        jnp.dot(b2_ref[...], Wc, preferred_element_type=f32, precision=lax.Precision.HIGHEST),
    ], axis=0)
    out_ref[...] = V


_weights = pl.pallas_call(
    _weights_body,
    out_shape=jax.ShapeDtypeStruct((4, N_CLASSES), jnp.float32),
)


def _combine_body(sums_ref, V_ref, bc_ref, out_ref):
    f32 = jnp.float32
    sums = sums_ref[...]
    cnt = sums[3:4, :]
    inv = 1.0 / jnp.maximum(cnt, 1.0)
    M4 = sums * inv
    out = lax.dot_general(M4, V_ref[...], (((0,), (0,)), ((), ())),
                          preferred_element_type=f32,
                          precision=lax.Precision.HIGHEST)
    out_ref[...] = out + bc_ref[...]


_combine = pl.pallas_call(
    _combine_body,
    out_shape=jax.ShapeDtypeStruct((N_GRAPHS, N_CLASSES), jnp.float32),
)


@jax.jit
def kernel(edge_index, node_graph_ids, W0, b0, W1, b1, W2, b2, Wc, bc):
    pad = jnp.full((2, EPAD - N_EDGES), PAD_NODE, jnp.int32)
    edges4 = jnp.concatenate([edge_index, pad], axis=1).reshape(2, NSUB, EC, CHUNK)
    gpad = jnp.full((NPAD - N_NODES,), PAD_GID, jnp.int32)
    gid3 = jnp.concatenate([node_graph_ids.astype(jnp.int32),
                            gpad]).reshape(NSUB, GC, GCHUNK)

    V = _weights(W0, b0.reshape(1, HIDDEN), W1, b1.reshape(1, HIDDEN),
                 W2, b2.reshape(1, HIDDEN), Wc)
    sums = _sc_sums(edges4, gid3)
    return _combine(sums, V, bc.reshape(1, N_CLASSES))


# speedup vs baseline: 1.1298x; 1.0032x over previous
import functools

import jax
import jax.numpy as jnp
from jax import lax
from jax.experimental import pallas as pl
from jax.experimental.pallas import tpu as pltpu
from jax.experimental.pallas import tpu_sc as plsc

N_NODES = 10000
N_EDGES = 160000
N_GRAPHS = 128
HIDDEN = 256
N_CLASSES = 10

NSUB = 16
NPAD = 10240
NT = NPAD // NSUB
CHUNK = 128
EC = 80
KW = 16
NB = EC // KW
EPT = EC * CHUNK
EPAD = NSUB * EPT
PAD_NODE = NPAD - 1
GCHUNK = 128
GC = NT // GCHUNK
PAD_GID = N_GRAPHS
GBUF = 136


def _rsqrt16(m):
    i = lax.bitcast_convert_type(m, jnp.int32)
    i = jnp.int32(0x5F3759DF) - (i >> 1)
    x = lax.bitcast_convert_type(i, jnp.float32)
    for _ in range(3):
        x = x * (1.5 - 0.5 * m * x * x)
    return x


def _sc_body(edges_hbm, gid_hbm, out_hbm,
             ya, aa, g0, g1, g2, g3,
             src_v, dst_v, gid_v, vals, ones_v, zeros_v,
             la, norm_v, n2_v, keep1, keep2, sem_g, sem_s):
    cid = lax.axis_index("c")
    tid = lax.axis_index("s")
    is0 = cid == 0

    one16 = jnp.full((16,), 1.0, jnp.float32)
    zero16 = jnp.zeros((16,), jnp.float32)

    def init_body(i, c):
        @pl.when(i < CHUNK // 16)
        def _():
            ones_v[pl.ds(i * 16, 16)] = one16
        zeros_v[pl.ds(i * 16, 16)] = zero16
        return c

    lax.fori_loop(0, NT // 16, init_body, 0)

    pltpu.sync_copy(edges_hbm.at[0, tid], src_v)
    pltpu.sync_copy(edges_hbm.at[1, tid], dst_v)
    pltpu.sync_copy(gid_hbm.at[tid], gid_v)

    nsl = pl.ds(tid * NT, NT)

    pltpu.sync_copy(zeros_v, aa.at[nsl])

    @pl.when(tid == 0)
    def _():
        z = zeros_v.at[pl.ds(0, GBUF)]
        pltpu.sync_copy(z, g0)
        pltpu.sync_copy(z, g1)
        pltpu.sync_copy(z, g2)
        pltpu.sync_copy(z, g3)

    plsc.subcore_barrier()

    def fire_gathers(b):
        def f(k, c):
            j = b * KW + k
            pltpu.async_copy(ya.at[src_v.at[j]], vals.at[j], sem_g)
            return c
        lax.fori_loop(0, KW, f, 0)

    def drain_gathers(b):
        def f(k, c):
            j = b * KW + k
            pltpu.make_async_copy(ya.at[src_v.at[j]], vals.at[j], sem_g).wait()
            return c
        lax.fori_loop(0, KW, f, 0)

    def fire_scatters(b, valrow):
        def f(k, c):
            j = b * KW + k
            pltpu.async_copy(valrow(j), aa.at[dst_v.at[j]], sem_s, add=True)
            return c
        lax.fori_loop(0, KW, f, 0)

    def drain_scatters(b, valrow):
        def f(k, c):
            j = b * KW + k
            pltpu.make_async_copy(valrow(j), aa.at[dst_v.at[j]], sem_s).wait()
            return c
        lax.fori_loop(0, KW, f, 0)

    ones_row = lambda j: ones_v

    def deg_blk(b, c):
        @pl.when(b > 0)
        def _():
            drain_scatters(b - 1, ones_row)
        fire_scatters(b, ones_row)
        return c

    lax.fori_loop(0, NB, deg_blk, 0)
    drain_scatters(NB - 1, ones_row)
    plsc.subcore_barrier()

    def edge_pass():
        vrow = lambda j: vals.at[j]
        fire_gathers(0)

        def blk(b, c):
            drain_gathers(b)

            @pl.when(b + 1 < NB)
            def _():
                fire_gathers(b + 1)

            @pl.when(b > 0)
            def _():
                drain_scatters(b - 1, vrow)

            fire_scatters(b, vrow)
            return c

        lax.fori_loop(0, NB, blk, 0)
        drain_scatters(NB - 1, vrow)

    pltpu.sync_copy(aa.at[nsl], la)

    def phaseB_body(i, c):
        s16 = pl.ds(i * 16, 16)
        d = la[s16]
        n = _rsqrt16(jnp.maximum(d, 1.0))
        norm_v[s16] = n
        n2_v[s16] = n * n
        la[s16] = jnp.where(is0, d * n, n)
        return c

    lax.fori_loop(0, NT // 16, phaseB_body, 0)
    pltpu.sync_copy(la, ya.at[nsl])
    pltpu.sync_copy(zeros_v, aa.at[nsl])
    plsc.subcore_barrier()

    edge_pass()
    plsc.subcore_barrier()

    pltpu.sync_copy(aa.at[nsl], la)

    def phaseC_body(i, c):
        s16 = pl.ds(i * 16, 16)
        v = la[s16]
        keep1[s16] = norm_v[s16] * v
        la[s16] = n2_v[s16] * v
        return c

    lax.fori_loop(0, NT // 16, phaseC_body, 0)
    pltpu.sync_copy(la, ya.at[nsl])
    pltpu.sync_copy(zeros_v, aa.at[nsl])
    plsc.subcore_barrier()

    edge_pass()
    plsc.subcore_barrier()

    pltpu.sync_copy(aa.at[nsl], la)

    def phaseD_body(i, c):
        s16 = pl.ds(i * 16, 16)
        v = la[s16]
        keep2[s16] = norm_v[s16] * v
        la[s16] = n2_v[s16] * v
        return c

    lax.fori_loop(0, NT // 16, phaseD_body, 0)

    @pl.when(is0)
    def _():
        pltpu.sync_copy(la, ya.at[nsl])
        pltpu.sync_copy(zeros_v, aa.at[nsl])
        plsc.subcore_barrier()
        edge_pass()
        plsc.subcore_barrier()
        pltpu.sync_copy(aa.at[nsl], la)

        def phaseE_body(i, c):
            s16 = pl.ds(i * 16, 16)
            keep2[s16] = norm_v[s16] * la[s16]
            return c

        lax.fori_loop(0, NT // 16, phaseE_body, 0)

    def pool_scatter(valref, gbuf):
        def body(j, c):
            off = pl.multiple_of(j * GCHUNK, GCHUNK)
            pltpu.sync_copy(valref.at[pl.ds(off, GCHUNK)],
                            gbuf.at[gid_v.at[j]], add=True)
            return c

        lax.fori_loop(0, GC, body, 0)

    @pl.when(is0)
    def _():
        pool_scatter(keep2, g0)

    @pl.when(jnp.logical_not(is0))
    def _():
        pool_scatter(keep2, g1)
        pool_scatter(keep1, g2)

        def cnt_body(j, c):
            pltpu.sync_copy(ones_v.at[pl.ds(0, GCHUNK)],
                            g3.at[gid_v.at[j]], add=True)
            return c

        lax.fori_loop(0, GC, cnt_body, 0)

    plsc.subcore_barrier()

    @pl.when(jnp.logical_and(is0, tid == 0))
    def _():
        pltpu.sync_copy(g0.at[pl.ds(0, N_GRAPHS)], out_hbm.at[0])

    @pl.when(jnp.logical_and(jnp.logical_not(is0), tid == 0))
    def _():
        pltpu.sync_copy(g1.at[pl.ds(0, N_GRAPHS)], out_hbm.at[1])
        pltpu.sync_copy(g2.at[pl.ds(0, N_GRAPHS)], out_hbm.at[2])
        pltpu.sync_copy(g3.at[pl.ds(0, N_GRAPHS)], out_hbm.at[3])


_sc_sums = pl.kernel(
    _sc_body,
    out_type=jax.ShapeDtypeStruct((4, N_GRAPHS), jnp.float32),
    mesh=plsc.VectorSubcoreMesh(core_axis_name="c", subcore_axis_name="s"),
    scratch_types=[
        pltpu.VMEM_SHARED((NPAD,), jnp.float32),
        pltpu.VMEM_SHARED((NPAD,), jnp.float32),
        pltpu.VMEM_SHARED((GBUF,), jnp.float32),
        pltpu.VMEM_SHARED((GBUF,), jnp.float32),
        pltpu.VMEM_SHARED((GBUF,), jnp.float32),
        pltpu.VMEM_SHARED((GBUF,), jnp.float32),
        pltpu.VMEM((EC, CHUNK), jnp.int32),
        pltpu.VMEM((EC, CHUNK), jnp.int32),
        pltpu.VMEM((GC, GCHUNK), jnp.int32),
        pltpu.VMEM((EC, CHUNK), jnp.float32),

        pltpu.VMEM((CHUNK,), jnp.float32),
        pltpu.VMEM((NT,), jnp.float32),
        pltpu.VMEM((NT,), jnp.float32),
        pltpu.VMEM((NT,), jnp.float32),
        pltpu.VMEM((NT,), jnp.float32),
        pltpu.VMEM((NT,), jnp.float32),
        pltpu.VMEM((NT,), jnp.float32),
        pltpu.SemaphoreType.DMA,
        pltpu.SemaphoreType.DMA,
    ],
)


def _weights_body(W0_ref, b0_ref, W1_ref, b1_ref, W2_ref, b2_ref,
                  Wc_ref, out_ref):
    f32 = jnp.float32
    W1 = W1_ref[...]
    W2 = W2_ref[...]
    Wc = Wc_ref[...]
    A = jnp.dot(jnp.dot(W0_ref[...], W1, preferred_element_type=f32, precision=lax.Precision.HIGHEST), W2,
                preferred_element_type=f32, precision=lax.Precision.HIGHEST)
    B = jnp.dot(jnp.dot(b0_ref[...], W1, preferred_element_type=f32, precision=lax.Precision.HIGHEST), W2,
                preferred_element_type=f32, precision=lax.Precision.HIGHEST)
    C = jnp.dot(b1_ref[...], W2, preferred_element_type=f32, precision=lax.Precision.HIGHEST)
    V = jnp.concatenate([
        jnp.dot(A, Wc, preferred_element_type=f32, precision=lax.Precision.HIGHEST),
        jnp.dot(B, Wc, preferred_element_type=f32, precision=lax.Precision.HIGHEST),
        jnp.dot(C, Wc, preferred_element_type=f32, precision=lax.Precision.HIGHEST),
        jnp.dot(b2_ref[...], Wc, preferred_element_type=f32, precision=lax.Precision.HIGHEST),
    ], axis=0)
    out_ref[...] = V


_weights = pl.pallas_call(
    _weights_body,
    out_shape=jax.ShapeDtypeStruct((4, N_CLASSES), jnp.float32),
)


def _combine_body(sums_ref, V_ref, bc_ref, out_ref):
    f32 = jnp.float32
    sums = sums_ref[...]
    cnt = sums[3:4, :]
    inv = 1.0 / jnp.maximum(cnt, 1.0)
    M4 = sums * inv
    out = lax.dot_general(M4, V_ref[...], (((0,), (0,)), ((), ())),
                          preferred_element_type=f32,
                          precision=lax.Precision.HIGHEST)
    out_ref[...] = out + bc_ref[...]


_combine = pl.pallas_call(
    _combine_body,
    out_shape=jax.ShapeDtypeStruct((N_GRAPHS, N_CLASSES), jnp.float32),
)


@jax.jit
def kernel(edge_index, node_graph_ids, W0, b0, W1, b1, W2, b2, Wc, bc):
    pad = jnp.full((2, EPAD - N_EDGES), PAD_NODE, jnp.int32)
    edges4 = jnp.concatenate([edge_index, pad], axis=1).reshape(2, NSUB, EC, CHUNK)
    gpad = jnp.full((NPAD - N_NODES,), PAD_GID, jnp.int32)
    gid3 = jnp.concatenate([node_graph_ids.astype(jnp.int32),
                            gpad]).reshape(NSUB, GC, GCHUNK)

    V = _weights(W0, b0.reshape(1, HIDDEN), W1, b1.reshape(1, HIDDEN),
                 W2, b2.reshape(1, HIDDEN), Wc)
    sums = _sc_sums(edges4, gid3)
    return _combine(sums, V, bc.reshape(1, N_CLASSES))


# speedup vs baseline: 1.1423x; 1.0111x over previous
import functools

import jax
import jax.numpy as jnp
from jax import lax
from jax.experimental import pallas as pl
from jax.experimental.pallas import tpu as pltpu
from jax.experimental.pallas import tpu_sc as plsc

N_NODES = 10000
N_EDGES = 160000
N_GRAPHS = 128
HIDDEN = 256
N_CLASSES = 10

NSUB = 16
NPAD = 10240
NT = NPAD // NSUB
CHUNK = 128
EC = 80
KW = 16
NB = EC // KW
EPT = EC * CHUNK
EPAD = NSUB * EPT
PAD_NODE = NPAD - 1
GCHUNK = 128
GC = NT // GCHUNK
PAD_GID = N_GRAPHS
GBUF = 136


def _rsqrt16(m):
    i = lax.bitcast_convert_type(m, jnp.int32)
    i = jnp.int32(0x5F3759DF) - (i >> 1)
    x = lax.bitcast_convert_type(i, jnp.float32)
    for _ in range(3):
        x = x * (1.5 - 0.5 * m * x * x)
    return x


def _sc_body(edges_hbm, gid_hbm, out_hbm,
             ya, aa, g0, g1, g2, g3,
             src_v, dst_v, gid_v, vals, ones_v, zeros_v,
             la, norm_v, n2_v, keep1, keep2, sem_g, sem_s):
    cid = lax.axis_index("c")
    tid = lax.axis_index("s")
    is0 = cid == 0

    one16 = jnp.full((16,), 1.0, jnp.float32)
    zero16 = jnp.zeros((16,), jnp.float32)

    def init_body(i, c):
        @pl.when(i < CHUNK // 16)
        def _():
            ones_v[pl.ds(i * 16, 16)] = one16
        zeros_v[pl.ds(i * 16, 16)] = zero16
        return c

    lax.fori_loop(0, NT // 16, init_body, 0)

    src_cp = pltpu.make_async_copy(edges_hbm.at[0, tid], src_v, sem_g)
    src_cp.start()
    pltpu.sync_copy(edges_hbm.at[1, tid], dst_v)
    pltpu.sync_copy(gid_hbm.at[tid], gid_v)

    nsl = pl.ds(tid * NT, NT)

    pltpu.sync_copy(zeros_v, aa.at[nsl])

    @pl.when(tid == 0)
    def _():
        z = zeros_v.at[pl.ds(0, GBUF)]
        pltpu.sync_copy(z, g0)
        pltpu.sync_copy(z, g1)
        pltpu.sync_copy(z, g2)
        pltpu.sync_copy(z, g3)

    plsc.subcore_barrier()

    def fire_gathers(b):
        def f(k, c):
            j = b * KW + k
            pltpu.async_copy(ya.at[src_v.at[j]], vals.at[j], sem_g)
            return c
        lax.fori_loop(0, KW, f, 0)

    def drain_gathers(b):
        def f(k, c):
            j = b * KW + k
            pltpu.make_async_copy(ya.at[src_v.at[j]], vals.at[j], sem_g).wait()
            return c
        lax.fori_loop(0, KW, f, 0)

    def fire_scatters(b, valrow):
        def f(k, c):
            j = b * KW + k
            pltpu.async_copy(valrow(j), aa.at[dst_v.at[j]], sem_s, add=True)
            return c
        lax.fori_loop(0, KW, f, 0)

    def drain_scatters(b, valrow):
        def f(k, c):
            j = b * KW + k
            pltpu.make_async_copy(valrow(j), aa.at[dst_v.at[j]], sem_s).wait()
            return c
        lax.fori_loop(0, KW, f, 0)

    ones_row = lambda j: ones_v

    def deg_blk(b, c):
        @pl.when(b > 0)
        def _():
            drain_scatters(b - 1, ones_row)
        fire_scatters(b, ones_row)
        return c

    lax.fori_loop(0, NB, deg_blk, 0)
    drain_scatters(NB - 1, ones_row)
    src_cp.wait()
    plsc.subcore_barrier()

    def edge_pass():
        vrow = lambda j: vals.at[j]
        fire_gathers(0)

        def blk(b, c):
            drain_gathers(b)

            @pl.when(b + 1 < NB)
            def _():
                fire_gathers(b + 1)

            @pl.when(b > 0)
            def _():
                drain_scatters(b - 1, vrow)

            fire_scatters(b, vrow)
            return c

        lax.fori_loop(0, NB, blk, 0)
        drain_scatters(NB - 1, vrow)

    pltpu.sync_copy(aa.at[nsl], la)

    def phaseB_body(i, c):
        s16 = pl.ds(i * 16, 16)
        d = la[s16]
        n = _rsqrt16(jnp.maximum(d, 1.0))
        norm_v[s16] = n
        n2_v[s16] = n * n
        la[s16] = jnp.where(is0, d * n, n)
        return c

    lax.fori_loop(0, NT // 16, phaseB_body, 0)
    pltpu.sync_copy(la, ya.at[nsl])
    pltpu.sync_copy(zeros_v, aa.at[nsl])
    plsc.subcore_barrier()

    edge_pass()
    plsc.subcore_barrier()

    pltpu.sync_copy(aa.at[nsl], la)

    def phaseC_body(i, c):
        s16 = pl.ds(i * 16, 16)
        v = la[s16]
        keep1[s16] = norm_v[s16] * v
        la[s16] = n2_v[s16] * v
        return c

    lax.fori_loop(0, NT // 16, phaseC_body, 0)
    pltpu.sync_copy(la, ya.at[nsl])
    pltpu.sync_copy(zeros_v, aa.at[nsl])
    plsc.subcore_barrier()

    edge_pass()
    plsc.subcore_barrier()

    pltpu.sync_copy(aa.at[nsl], la)

    def phaseD_body(i, c):
        s16 = pl.ds(i * 16, 16)
        v = la[s16]
        keep2[s16] = norm_v[s16] * v
        la[s16] = n2_v[s16] * v
        return c

    lax.fori_loop(0, NT // 16, phaseD_body, 0)

    @pl.when(is0)
    def _():
        pltpu.sync_copy(la, ya.at[nsl])
        pltpu.sync_copy(zeros_v, aa.at[nsl])
        plsc.subcore_barrier()
        edge_pass()
        plsc.subcore_barrier()
        pltpu.sync_copy(aa.at[nsl], la)

        def phaseE_body(i, c):
            s16 = pl.ds(i * 16, 16)
            keep2[s16] = norm_v[s16] * la[s16]
            return c

        lax.fori_loop(0, NT // 16, phaseE_body, 0)

    def pool_scatter(valref, gbuf):
        def body(j, c):
            off = pl.multiple_of(j * GCHUNK, GCHUNK)
            pltpu.sync_copy(valref.at[pl.ds(off, GCHUNK)],
                            gbuf.at[gid_v.at[j]], add=True)
            return c

        lax.fori_loop(0, GC, body, 0)

    @pl.when(is0)
    def _():
        pool_scatter(keep2, g0)

    @pl.when(jnp.logical_not(is0))
    def _():
        pool_scatter(keep2, g1)
        pool_scatter(keep1, g2)

        def cnt_body(j, c):
            pltpu.sync_copy(ones_v.at[pl.ds(0, GCHUNK)],
                            g3.at[gid_v.at[j]], add=True)
            return c

        lax.fori_loop(0, GC, cnt_body, 0)

    plsc.subcore_barrier()

    @pl.when(jnp.logical_and(is0, tid == 0))
    def _():
        pltpu.sync_copy(g0.at[pl.ds(0, N_GRAPHS)], out_hbm.at[0])

    @pl.when(jnp.logical_and(jnp.logical_not(is0), tid == 0))
    def _():
        pltpu.sync_copy(g1.at[pl.ds(0, N_GRAPHS)], out_hbm.at[1])
        pltpu.sync_copy(g2.at[pl.ds(0, N_GRAPHS)], out_hbm.at[2])
        pltpu.sync_copy(g3.at[pl.ds(0, N_GRAPHS)], out_hbm.at[3])


_sc_sums = pl.kernel(
    _sc_body,
    out_type=jax.ShapeDtypeStruct((4, N_GRAPHS), jnp.float32),
    mesh=plsc.VectorSubcoreMesh(core_axis_name="c", subcore_axis_name="s"),
    scratch_types=[
        pltpu.VMEM_SHARED((NPAD,), jnp.float32),
        pltpu.VMEM_SHARED((NPAD,), jnp.float32),
        pltpu.VMEM_SHARED((GBUF,), jnp.float32),
        pltpu.VMEM_SHARED((GBUF,), jnp.float32),
        pltpu.VMEM_SHARED((GBUF,), jnp.float32),
        pltpu.VMEM_SHARED((GBUF,), jnp.float32),
        pltpu.VMEM((EC, CHUNK), jnp.int32),
        pltpu.VMEM((EC, CHUNK), jnp.int32),
        pltpu.VMEM((GC, GCHUNK), jnp.int32),
        pltpu.VMEM((EC, CHUNK), jnp.float32),

        pltpu.VMEM((CHUNK,), jnp.float32),
        pltpu.VMEM((NT,), jnp.float32),
        pltpu.VMEM((NT,), jnp.float32),
        pltpu.VMEM((NT,), jnp.float32),
        pltpu.VMEM((NT,), jnp.float32),
        pltpu.VMEM((NT,), jnp.float32),
        pltpu.VMEM((NT,), jnp.float32),
        pltpu.SemaphoreType.DMA,
        pltpu.SemaphoreType.DMA,
    ],
)


def _weights_body(W0_ref, b0_ref, W1_ref, b1_ref, W2_ref, b2_ref,
                  Wc_ref, out_ref):
    f32 = jnp.float32
    W1 = W1_ref[...]
    W2 = W2_ref[...]
    Wc = Wc_ref[...]
    A = jnp.dot(jnp.dot(W0_ref[...], W1, preferred_element_type=f32, precision=lax.Precision.HIGHEST), W2,
                preferred_element_type=f32, precision=lax.Precision.HIGHEST)
    B = jnp.dot(jnp.dot(b0_ref[...], W1, preferred_element_type=f32, precision=lax.Precision.HIGHEST), W2,
                preferred_element_type=f32, precision=lax.Precision.HIGHEST)
    C = jnp.dot(b1_ref[...], W2, preferred_element_type=f32, precision=lax.Precision.HIGHEST)
    V = jnp.concatenate([
        jnp.dot(A, Wc, preferred_element_type=f32, precision=lax.Precision.HIGHEST),
        jnp.dot(B, Wc, preferred_element_type=f32, precision=lax.Precision.HIGHEST),
        jnp.dot(C, Wc, preferred_element_type=f32, precision=lax.Precision.HIGHEST),
        jnp.dot(b2_ref[...], Wc, preferred_element_type=f32, precision=lax.Precision.HIGHEST),
    ], axis=0)
    out_ref[...] = V


_weights = pl.pallas_call(
    _weights_body,
    out_shape=jax.ShapeDtypeStruct((4, N_CLASSES), jnp.float32),
)


def _combine_body(sums_ref, V_ref, bc_ref, out_ref):
    f32 = jnp.float32
    sums = sums_ref[...]
    cnt = sums[3:4, :]
    inv = 1.0 / jnp.maximum(cnt, 1.0)
    M4 = sums * inv
    out = lax.dot_general(M4, V_ref[...], (((0,), (0,)), ((), ())),
                          preferred_element_type=f32,
                          precision=lax.Precision.HIGHEST)
    out_ref[...] = out + bc_ref[...]


_combine = pl.pallas_call(
    _combine_body,
    out_shape=jax.ShapeDtypeStruct((N_GRAPHS, N_CLASSES), jnp.float32),
)


@jax.jit
def kernel(edge_index, node_graph_ids, W0, b0, W1, b1, W2, b2, Wc, bc):
    pad = jnp.full((2, EPAD - N_EDGES), PAD_NODE, jnp.int32)
    edges4 = jnp.concatenate([edge_index, pad], axis=1).reshape(2, NSUB, EC, CHUNK)
    gpad = jnp.full((NPAD - N_NODES,), PAD_GID, jnp.int32)
    gid3 = jnp.concatenate([node_graph_ids.astype(jnp.int32),
                            gpad]).reshape(NSUB, GC, GCHUNK)

    V = _weights(W0, b0.reshape(1, HIDDEN), W1, b1.reshape(1, HIDDEN),
                 W2, b2.reshape(1, HIDDEN), Wc)
    sums = _sc_sums(edges4, gid3)
    return _combine(sums, V, bc.reshape(1, N_CLASSES))


# speedup vs baseline: 1.1562x; 1.0121x over previous
import functools

import jax
import jax.numpy as jnp
from jax import lax
from jax.experimental import pallas as pl
from jax.experimental.pallas import tpu as pltpu
from jax.experimental.pallas import tpu_sc as plsc

N_NODES = 10000
N_EDGES = 160000
N_GRAPHS = 128
HIDDEN = 256
N_CLASSES = 10

NSUB = 16
NPAD = 10240
NT = NPAD // NSUB
CHUNK = 128
EC = 80
KW = 8
NB = EC // KW
EPT = EC * CHUNK
EPAD = NSUB * EPT
PAD_NODE = NPAD - 1
GCHUNK = 128
GC = NT // GCHUNK
PAD_GID = N_GRAPHS
GBUF = 136


def _rsqrt16(m):
    i = lax.bitcast_convert_type(m, jnp.int32)
    i = jnp.int32(0x5F3759DF) - (i >> 1)
    x = lax.bitcast_convert_type(i, jnp.float32)
    for _ in range(3):
        x = x * (1.5 - 0.5 * m * x * x)
    return x


def _sc_body(edges_hbm, gid_hbm, out_hbm,
             ya, aa, g0, g1, g2, g3,
             src_v, dst_v, gid_v, vals, ones_v, zeros_v,
             la, norm_v, n2_v, keep1, keep2, sem_g, sem_s):
    cid = lax.axis_index("c")
    tid = lax.axis_index("s")
    is0 = cid == 0

    one16 = jnp.full((16,), 1.0, jnp.float32)
    zero16 = jnp.zeros((16,), jnp.float32)

    def init_body(i, c):
        @pl.when(i < CHUNK // 16)
        def _():
            ones_v[pl.ds(i * 16, 16)] = one16
        zeros_v[pl.ds(i * 16, 16)] = zero16
        return c

    lax.fori_loop(0, NT // 16, init_body, 0)

    src_cp = pltpu.make_async_copy(edges_hbm.at[0, tid], src_v, sem_g)
    src_cp.start()
    pltpu.sync_copy(edges_hbm.at[1, tid], dst_v)
    pltpu.sync_copy(gid_hbm.at[tid], gid_v)

    nsl = pl.ds(tid * NT, NT)

    pltpu.sync_copy(zeros_v, aa.at[nsl])

    @pl.when(tid == 0)
    def _():
        z = zeros_v.at[pl.ds(0, GBUF)]
        pltpu.sync_copy(z, g0)
        pltpu.sync_copy(z, g1)
        pltpu.sync_copy(z, g2)
        pltpu.sync_copy(z, g3)

    plsc.subcore_barrier()

    def fire_gathers(b):
        def f(k, c):
            j = b * KW + k
            pltpu.async_copy(ya.at[src_v.at[j]], vals.at[j], sem_g)
            return c
        lax.fori_loop(0, KW, f, 0)

    def drain_gathers(b):
        def f(k, c):
            j = b * KW + k
            pltpu.make_async_copy(ya.at[src_v.at[j]], vals.at[j], sem_g).wait()
            return c
        lax.fori_loop(0, KW, f, 0)

    def fire_scatters(b, valrow):
        def f(k, c):
            j = b * KW + k
            pltpu.async_copy(valrow(j), aa.at[dst_v.at[j]], sem_s, add=True)
            return c
        lax.fori_loop(0, KW, f, 0)

    def drain_scatters(b, valrow):
        def f(k, c):
            j = b * KW + k
            pltpu.make_async_copy(valrow(j), aa.at[dst_v.at[j]], sem_s).wait()
            return c
        lax.fori_loop(0, KW, f, 0)

    ones_row = lambda j: ones_v

    def deg_blk(b, c):
        @pl.when(b > 0)
        def _():
            drain_scatters(b - 1, ones_row)
        fire_scatters(b, ones_row)
        return c

    lax.fori_loop(0, NB, deg_blk, 0)
    drain_scatters(NB - 1, ones_row)
    src_cp.wait()
    plsc.subcore_barrier()

    def edge_pass():
        vrow = lambda j: vals.at[j]
        fire_gathers(0)

        def blk(b, c):
            drain_gathers(b)

            @pl.when(b + 1 < NB)
            def _():
                fire_gathers(b + 1)

            @pl.when(b > 0)
            def _():
                drain_scatters(b - 1, vrow)

            fire_scatters(b, vrow)
            return c

        lax.fori_loop(0, NB, blk, 0)
        drain_scatters(NB - 1, vrow)

    pltpu.sync_copy(aa.at[nsl], la)

    def phaseB_body(i, c):
        s16 = pl.ds(i * 16, 16)
        d = la[s16]
        n = _rsqrt16(jnp.maximum(d, 1.0))
        norm_v[s16] = n
        n2_v[s16] = n * n
        la[s16] = jnp.where(is0, d * n, n)
        return c

    lax.fori_loop(0, NT // 16, phaseB_body, 0)
    pltpu.sync_copy(la, ya.at[nsl])
    pltpu.sync_copy(zeros_v, aa.at[nsl])
    plsc.subcore_barrier()

    edge_pass()
    plsc.subcore_barrier()

    pltpu.sync_copy(aa.at[nsl], la)

    def phaseC_body(i, c):
        s16 = pl.ds(i * 16, 16)
        v = la[s16]
        keep1[s16] = norm_v[s16] * v
        la[s16] = n2_v[s16] * v
        return c

    lax.fori_loop(0, NT // 16, phaseC_body, 0)
    pltpu.sync_copy(la, ya.at[nsl])
    pltpu.sync_copy(zeros_v, aa.at[nsl])
    plsc.subcore_barrier()

    edge_pass()
    plsc.subcore_barrier()

    pltpu.sync_copy(aa.at[nsl], la)

    def phaseD_body(i, c):
        s16 = pl.ds(i * 16, 16)
        v = la[s16]
        keep2[s16] = norm_v[s16] * v
        la[s16] = n2_v[s16] * v
        return c

    lax.fori_loop(0, NT // 16, phaseD_body, 0)

    @pl.when(is0)
    def _():
        pltpu.sync_copy(la, ya.at[nsl])
        pltpu.sync_copy(zeros_v, aa.at[nsl])
        plsc.subcore_barrier()
        edge_pass()
        plsc.subcore_barrier()
        pltpu.sync_copy(aa.at[nsl], la)

        def phaseE_body(i, c):
            s16 = pl.ds(i * 16, 16)
            keep2[s16] = norm_v[s16] * la[s16]
            return c

        lax.fori_loop(0, NT // 16, phaseE_body, 0)

    def pool_scatter(valref, gbuf):
        def body(j, c):
            off = pl.multiple_of(j * GCHUNK, GCHUNK)
            pltpu.sync_copy(valref.at[pl.ds(off, GCHUNK)],
                            gbuf.at[gid_v.at[j]], add=True)
            return c

        lax.fori_loop(0, GC, body, 0)

    @pl.when(is0)
    def _():
        pool_scatter(keep2, g0)

    @pl.when(jnp.logical_not(is0))
    def _():
        pool_scatter(keep2, g1)
        pool_scatter(keep1, g2)

        def cnt_body(j, c):
            pltpu.sync_copy(ones_v.at[pl.ds(0, GCHUNK)],
                            g3.at[gid_v.at[j]], add=True)
            return c

        lax.fori_loop(0, GC, cnt_body, 0)

    plsc.subcore_barrier()

    @pl.when(jnp.logical_and(is0, tid == 0))
    def _():
        pltpu.sync_copy(g0.at[pl.ds(0, N_GRAPHS)], out_hbm.at[0])

    @pl.when(jnp.logical_and(jnp.logical_not(is0), tid == 0))
    def _():
        pltpu.sync_copy(g1.at[pl.ds(0, N_GRAPHS)], out_hbm.at[1])
        pltpu.sync_copy(g2.at[pl.ds(0, N_GRAPHS)], out_hbm.at[2])
        pltpu.sync_copy(g3.at[pl.ds(0, N_GRAPHS)], out_hbm.at[3])


_sc_sums = pl.kernel(
    _sc_body,
    out_type=jax.ShapeDtypeStruct((4, N_GRAPHS), jnp.float32),
    mesh=plsc.VectorSubcoreMesh(core_axis_name="c", subcore_axis_name="s"),
    scratch_types=[
        pltpu.VMEM_SHARED((NPAD,), jnp.float32),
        pltpu.VMEM_SHARED((NPAD,), jnp.float32),
        pltpu.VMEM_SHARED((GBUF,), jnp.float32),
        pltpu.VMEM_SHARED((GBUF,), jnp.float32),
        pltpu.VMEM_SHARED((GBUF,), jnp.float32),
        pltpu.VMEM_SHARED((GBUF,), jnp.float32),
        pltpu.VMEM((EC, CHUNK), jnp.int32),
        pltpu.VMEM((EC, CHUNK), jnp.int32),
        pltpu.VMEM((GC, GCHUNK), jnp.int32),
        pltpu.VMEM((EC, CHUNK), jnp.float32),

        pltpu.VMEM((CHUNK,), jnp.float32),
        pltpu.VMEM((NT,), jnp.float32),
        pltpu.VMEM((NT,), jnp.float32),
        pltpu.VMEM((NT,), jnp.float32),
        pltpu.VMEM((NT,), jnp.float32),
        pltpu.VMEM((NT,), jnp.float32),
        pltpu.VMEM((NT,), jnp.float32),
        pltpu.SemaphoreType.DMA,
        pltpu.SemaphoreType.DMA,
    ],
)


def _weights_body(W0_ref, b0_ref, W1_ref, b1_ref, W2_ref, b2_ref,
                  Wc_ref, out_ref):
    f32 = jnp.float32
    W1 = W1_ref[...]
    W2 = W2_ref[...]
    Wc = Wc_ref[...]
    A = jnp.dot(jnp.dot(W0_ref[...], W1, preferred_element_type=f32, precision=lax.Precision.HIGHEST), W2,
                preferred_element_type=f32, precision=lax.Precision.HIGHEST)
    B = jnp.dot(jnp.dot(b0_ref[...], W1, preferred_element_type=f32, precision=lax.Precision.HIGHEST), W2,
                preferred_element_type=f32, precision=lax.Precision.HIGHEST)
    C = jnp.dot(b1_ref[...], W2, preferred_element_type=f32, precision=lax.Precision.HIGHEST)
    V = jnp.concatenate([
        jnp.dot(A, Wc, preferred_element_type=f32, precision=lax.Precision.HIGHEST),
        jnp.dot(B, Wc, preferred_element_type=f32, precision=lax.Precision.HIGHEST),
        jnp.dot(C, Wc, preferred_element_type=f32, precision=lax.Precision.HIGHEST),
        jnp.dot(b2_ref[...], Wc, preferred_element_type=f32, precision=lax.Precision.HIGHEST),
    ], axis=0)
    out_ref[...] = V


_weights = pl.pallas_call(
    _weights_body,
    out_shape=jax.ShapeDtypeStruct((4, N_CLASSES), jnp.float32),
)


def _combine_body(sums_ref, V_ref, bc_ref, out_ref):
    f32 = jnp.float32
    sums = sums_ref[...]
    cnt = sums[3:4, :]
    inv = 1.0 / jnp.maximum(cnt, 1.0)
    M4 = sums * inv
    out = lax.dot_general(M4, V_ref[...], (((0,), (0,)), ((), ())),
                          preferred_element_type=f32,
                          precision=lax.Precision.HIGHEST)
    out_ref[...] = out + bc_ref[...]


_combine = pl.pallas_call(
    _combine_body,
    out_shape=jax.ShapeDtypeStruct((N_GRAPHS, N_CLASSES), jnp.float32),
)


@jax.jit
def kernel(edge_index, node_graph_ids, W0, b0, W1, b1, W2, b2, Wc, bc):
    pad = jnp.full((2, EPAD - N_EDGES), PAD_NODE, jnp.int32)
    edges4 = jnp.concatenate([edge_index, pad], axis=1).reshape(2, NSUB, EC, CHUNK)
    gpad = jnp.full((NPAD - N_NODES,), PAD_GID, jnp.int32)
    gid3 = jnp.concatenate([node_graph_ids.astype(jnp.int32),
                            gpad]).reshape(NSUB, GC, GCHUNK)

    V = _weights(W0, b0.reshape(1, HIDDEN), W1, b1.reshape(1, HIDDEN),
                 W2, b2.reshape(1, HIDDEN), Wc)
    sums = _sc_sums(edges4, gid3)
    return _combine(sums, V, bc.reshape(1, N_CLASSES))
